# Initial kernel scaffold; baseline (speedup 1.0000x reference)
import jax
import jax.numpy as jnp
from jax.experimental import pallas as pl


def kernel(x, edge_index, enc_w1, enc_b1, enc_w2, enc_b2, s1_wl, s1_bl, s1_wr, s2_wl, s2_bl, s2_wr, ro_w1, ro_b1, ro_w2, ro_b2):
    raise NotImplementedError("write your pallas kernel here")


# speedup vs baseline: 2.1968x; 2.1968x over previous
import functools

import jax
import jax.numpy as jnp
from jax import lax
from jax.experimental import pallas as pl
from jax.experimental.pallas import tpu as pltpu
from jax.experimental.pallas import tpu_sc as plsc

N = 50000
NPAD = 50176
IN_DIM = 24
HID = 256
OUT_DIM = 12
E = 800000
EPAD = 819200
NC = 2
NS = 16
CW = 16
NCH = HID // CW
EPT = EPAD // NS
BB = 1024
NBATCH = EPT // BB
STRIPE = NPAD // NS
ZR = STRIPE // 4


def _dot(a, b):
    return lax.dot_general(a, b, (((1,), (0,)), ((), ())),
                           preferred_element_type=jnp.float32)


_BR = 1024
_GRID = NPAD // _BR


def _enc_body(x_ref, w1_ref, b1_ref, w2_ref, b2_ref, o_ref):
    h = jnp.maximum(_dot(x_ref[...], w1_ref[...]) + b1_ref[...], 0.0)
    o_ref[...] = jnp.maximum(_dot(h, w2_ref[...]) + b2_ref[...], 0.0)


def _encoder(xp, w1, b1, w2, b2):
    return pl.pallas_call(
        _enc_body,
        grid=(_GRID,),
        in_specs=[
            pl.BlockSpec((_BR, IN_DIM), lambda i: (i, 0)),
            pl.BlockSpec((IN_DIM, HID // 2), lambda i: (0, 0)),
            pl.BlockSpec((1, HID // 2), lambda i: (0, 0)),
            pl.BlockSpec((HID // 2, HID), lambda i: (0, 0)),
            pl.BlockSpec((1, HID), lambda i: (0, 0)),
        ],
        out_specs=pl.BlockSpec((_BR, HID), lambda i: (i, 0)),
        out_shape=jax.ShapeDtypeStruct((NPAD, HID), jnp.float32),
    )(xp, w1, b1.reshape(1, -1), w2, b2.reshape(1, -1))


def _combine_body(agg_ref, cnt_ref, h_ref, wl_ref, bl_ref, wr_ref, o_ref):
    inv = 1.0 / jnp.maximum(cnt_ref[...], 1.0)
    mean = agg_ref[...] * inv
    o_ref[...] = jnp.maximum(
        _dot(mean, wl_ref[...]) + bl_ref[...] + _dot(h_ref[...], wr_ref[...]),
        0.0)


def _combine(agg, cnt, h, wl, bl, wr):
    return pl.pallas_call(
        _combine_body,
        grid=(_GRID,),
        in_specs=[
            pl.BlockSpec((_BR, HID), lambda i: (i, 0)),
            pl.BlockSpec((_BR, 1), lambda i: (i, 0)),
            pl.BlockSpec((_BR, HID), lambda i: (i, 0)),
            pl.BlockSpec((HID, HID), lambda i: (0, 0)),
            pl.BlockSpec((1, HID), lambda i: (0, 0)),
            pl.BlockSpec((HID, HID), lambda i: (0, 0)),
        ],
        out_specs=pl.BlockSpec((_BR, HID), lambda i: (i, 0)),
        out_shape=jax.ShapeDtypeStruct((NPAD, HID), jnp.float32),
    )(agg, cnt, h, wl, bl.reshape(1, -1), wr)


def _combine_ro_body(agg_ref, cnt_ref, h_ref, wl_ref, bl_ref, wr_ref,
                     rw1_ref, rb1_ref, rw2_ref, rb2_ref, o_ref):
    inv = 1.0 / jnp.maximum(cnt_ref[...], 1.0)
    mean = agg_ref[...] * inv
    t = jnp.maximum(
        _dot(mean, wl_ref[...]) + bl_ref[...] + _dot(h_ref[...], wr_ref[...]),
        0.0)
    t = jnp.maximum(_dot(t, rw1_ref[...]) + rb1_ref[...], 0.0)
    o_ref[...] = _dot(t, rw2_ref[...]) + rb2_ref[...]


def _combine_readout(agg, cnt, h, wl, bl, wr, rw1, rb1, rw2, rb2):
    return pl.pallas_call(
        _combine_ro_body,
        grid=(_GRID,),
        in_specs=[
            pl.BlockSpec((_BR, HID), lambda i: (i, 0)),
            pl.BlockSpec((_BR, 1), lambda i: (i, 0)),
            pl.BlockSpec((_BR, HID), lambda i: (i, 0)),
            pl.BlockSpec((HID, HID), lambda i: (0, 0)),
            pl.BlockSpec((1, HID), lambda i: (0, 0)),
            pl.BlockSpec((HID, HID), lambda i: (0, 0)),
            pl.BlockSpec((HID, HID // 2), lambda i: (0, 0)),
            pl.BlockSpec((1, HID // 2), lambda i: (0, 0)),
            pl.BlockSpec((HID // 2, OUT_DIM), lambda i: (0, 0)),
            pl.BlockSpec((1, OUT_DIM), lambda i: (0, 0)),
        ],
        out_specs=pl.BlockSpec((_BR, OUT_DIM), lambda i: (i, 0)),
        out_shape=jax.ShapeDtypeStruct((NPAD, OUT_DIM), jnp.float32),
    )(agg, cnt, h, wl, bl.reshape(1, -1), wr,
      rw1, rb1.reshape(1, -1), rw2, rb2.reshape(1, -1))


@functools.cache
def _make_sc_kernel(with_count):
    mesh = plsc.VectorSubcoreMesh(core_axis_name="c", subcore_axis_name="s",
                                  num_cores=NC, num_subcores=NS)
    out_type = [jax.ShapeDtypeStruct((NPAD, HID), jnp.float32)]
    if with_count:
        out_type.append(jax.ShapeDtypeStruct((NPAD, CW), jnp.float32))

    scratch = [
        pltpu.VMEM((BB // 128, 128), jnp.int32),
        pltpu.VMEM((BB // 128, 128), jnp.int32),
        pltpu.VMEM((BB, CW), jnp.float32),
        pltpu.VMEM((ZR, CW), jnp.float32),
        pltpu.VMEM((128, CW), jnp.float32),
        pltpu.VMEM_SHARED((NPAD, CW), jnp.float32),
        pltpu.SemaphoreType.DMA,
    ]

    def body(h8, idx8, dstr, zeros_h, ones_h, *rest):
        if with_count:
            agg_out, cnt_out = rest[0], rest[1]
            idx_v, dst_v, rows_v, zb_v, ones_v, slab, gsem = rest[2:]
        else:
            agg_out = rest[0]
            cnt_out = None
            idx_v, dst_v, rows_v, zb_v, ones_v, slab, gsem = rest[1:]

        cid = lax.axis_index("c")
        sid = lax.axis_index("s")
        stripe0 = sid * STRIPE

        pltpu.sync_copy(zeros_h, zb_v)
        pltpu.sync_copy(ones_h, ones_v)

        def zero_slab():
            for z in range(4):
                pltpu.sync_copy(zb_v, slab.at[pl.ds(stripe0 + z * ZR, ZR), :])

        def gather_pass(ch):
            zero_slab()
            plsc.subcore_barrier()

            def batch(b, carry):
                row0 = sid * (EPT // 128) + b * (BB // 128)
                pltpu.sync_copy(dstr.at[pl.ds(row0, BB // 128), :], dst_v)
                pltpu.sync_copy(idx8.at[ch, pl.ds(row0, BB // 128), :], idx_v)
                handles = []
                for j in range(BB // 128):
                    handles.append(pltpu.async_copy(
                        h8.at[idx_v.at[j]],
                        rows_v.at[pl.ds(j * 128, 128), :], gsem))
                for hnd in handles:
                    hnd.wait()
                for j in range(BB // 128):
                    pltpu.sync_copy(rows_v.at[pl.ds(j * 128, 128), :],
                                    slab.at[dst_v.at[j]], add=True)
                return carry

            lax.fori_loop(0, NBATCH, batch, 0)
            plsc.subcore_barrier()
            pltpu.sync_copy(
                slab.at[pl.ds(stripe0, STRIPE), :],
                agg_out.at[pl.ds(stripe0, STRIPE), pl.ds(ch * CW, CW)])
            plsc.subcore_barrier()

        for j in range(NCH // NC):
            gather_pass(cid * (NCH // NC) + j)

        if with_count:
            @pl.when(cid == 0)
            def _count_pass():
                zero_slab()
                plsc.subcore_barrier()

                def batch(b, carry):
                    row0 = sid * (EPT // 128) + b * (BB // 128)
                    pltpu.sync_copy(dstr.at[pl.ds(row0, BB // 128), :], dst_v)
                    for j in range(BB // 128):
                        pltpu.sync_copy(ones_v, slab.at[dst_v.at[j]],
                                        add=True)
                    return carry

                lax.fori_loop(0, NBATCH, batch, 0)
                plsc.subcore_barrier()
                pltpu.sync_copy(slab.at[pl.ds(stripe0, STRIPE), :],
                                cnt_out.at[pl.ds(stripe0, STRIPE), :])

    return pl.kernel(
        body,
        out_type=tuple(out_type) if with_count else out_type[0],
        mesh=mesh,
        scratch_types=scratch,
        compiler_params=pltpu.CompilerParams(use_tc_tiling_on_sc=False),
    )


def kernel(x, edge_index, enc_w1, enc_b1, enc_w2, enc_b2,
           s1_wl, s1_bl, s1_wr, s2_wl, s2_bl, s2_wr,
           ro_w1, ro_b1, ro_w2, ro_b2):
    xp = jnp.pad(x, ((0, NPAD - N), (0, 0)))

    src = edge_index[0]
    dst = edge_index[1]
    src_p = jnp.concatenate(
        [src, jnp.zeros((EPAD - E,), dtype=jnp.int32)])
    dst_p = jnp.concatenate(
        [dst, jnp.full((EPAD - E,), N, dtype=jnp.int32)])
    idx8 = (src_p[None, :] * NCH
            + jnp.arange(NCH, dtype=jnp.int32)[:, None]).reshape(
                NCH, EPAD // 128, 128)
    dstr = dst_p.reshape(EPAD // 128, 128)
    zeros_h = jnp.zeros((ZR, CW), dtype=jnp.float32)
    ones_h = jnp.ones((128, CW), dtype=jnp.float32)

    h = _encoder(xp, enc_w1, enc_b1, enc_w2, enc_b2)

    agg1, cnt8 = _make_sc_kernel(True)(h.reshape(NPAD * NCH, CW), idx8, dstr,
                                       zeros_h, ones_h)
    cnt = cnt8[:, :1]

    h1 = _combine(agg1, cnt, h, s1_wl, s1_bl, s1_wr)

    agg2 = _make_sc_kernel(False)(h1.reshape(NPAD * NCH, CW), idx8, dstr,
                                  zeros_h, ones_h)

    out = _combine_readout(agg2, cnt, h1, s2_wl, s2_bl, s2_wr,
                           ro_w1, ro_b1, ro_w2, ro_b2)
    return out[:N]


# speedup vs baseline: 2.5657x; 1.1679x over previous
import functools

import jax
import jax.numpy as jnp
from jax import lax
from jax.experimental import pallas as pl
from jax.experimental.pallas import tpu as pltpu
from jax.experimental.pallas import tpu_sc as plsc

N = 50000
NPAD = 50176
IN_DIM = 24
HID = 256
OUT_DIM = 12
E = 800000
EPAD = 819200
NC = 2
NS = 16
CW = 32
NCH = HID // CW
EPT = EPAD // NS
BB = 512
SUB = BB // 128
NBATCH = EPT // BB
STRIPE = NPAD // NS


def _dot(a, b):
    return lax.dot_general(a, b, (((1,), (0,)), ((), ())),
                           preferred_element_type=jnp.float32)


_BR = 1024
_GRID = NPAD // _BR


def _enc_body(x_ref, w1_ref, b1_ref, w2_ref, b2_ref, o_ref):
    h = jnp.maximum(_dot(x_ref[...], w1_ref[...]) + b1_ref[...], 0.0)
    o_ref[...] = jnp.maximum(_dot(h, w2_ref[...]) + b2_ref[...], 0.0)


def _encoder(xp, w1, b1, w2, b2):
    return pl.pallas_call(
        _enc_body,
        grid=(_GRID,),
        in_specs=[
            pl.BlockSpec((_BR, IN_DIM), lambda i: (i, 0)),
            pl.BlockSpec((IN_DIM, HID // 2), lambda i: (0, 0)),
            pl.BlockSpec((1, HID // 2), lambda i: (0, 0)),
            pl.BlockSpec((HID // 2, HID), lambda i: (0, 0)),
            pl.BlockSpec((1, HID), lambda i: (0, 0)),
        ],
        out_specs=pl.BlockSpec((_BR, HID), lambda i: (i, 0)),
        out_shape=jax.ShapeDtypeStruct((NPAD, HID), jnp.float32),
    )(xp, w1, b1.reshape(1, -1), w2, b2.reshape(1, -1))


def _combine_body(agg_ref, cnt_ref, h_ref, wl_ref, bl_ref, wr_ref, o_ref):
    inv = 1.0 / jnp.maximum(cnt_ref[...], 1.0)
    mean = agg_ref[...] * inv
    o_ref[...] = jnp.maximum(
        _dot(mean, wl_ref[...]) + bl_ref[...] + _dot(h_ref[...], wr_ref[...]),
        0.0)


def _combine(agg, cnt, h, wl, bl, wr):
    return pl.pallas_call(
        _combine_body,
        grid=(_GRID,),
        in_specs=[
            pl.BlockSpec((_BR, HID), lambda i: (i, 0)),
            pl.BlockSpec((_BR, 1), lambda i: (i, 0)),
            pl.BlockSpec((_BR, HID), lambda i: (i, 0)),
            pl.BlockSpec((HID, HID), lambda i: (0, 0)),
            pl.BlockSpec((1, HID), lambda i: (0, 0)),
            pl.BlockSpec((HID, HID), lambda i: (0, 0)),
        ],
        out_specs=pl.BlockSpec((_BR, HID), lambda i: (i, 0)),
        out_shape=jax.ShapeDtypeStruct((NPAD, HID), jnp.float32),
    )(agg, cnt, h, wl, bl.reshape(1, -1), wr)


def _combine_ro_body(agg_ref, cnt_ref, h_ref, wl_ref, bl_ref, wr_ref,
                     rw1_ref, rb1_ref, rw2_ref, rb2_ref, o_ref):
    inv = 1.0 / jnp.maximum(cnt_ref[...], 1.0)
    mean = agg_ref[...] * inv
    t = jnp.maximum(
        _dot(mean, wl_ref[...]) + bl_ref[...] + _dot(h_ref[...], wr_ref[...]),
        0.0)
    t = jnp.maximum(_dot(t, rw1_ref[...]) + rb1_ref[...], 0.0)
    o_ref[...] = _dot(t, rw2_ref[...]) + rb2_ref[...]


def _combine_readout(agg, cnt, h, wl, bl, wr, rw1, rb1, rw2, rb2):
    return pl.pallas_call(
        _combine_ro_body,
        grid=(_GRID,),
        in_specs=[
            pl.BlockSpec((_BR, HID), lambda i: (i, 0)),
            pl.BlockSpec((_BR, 1), lambda i: (i, 0)),
            pl.BlockSpec((_BR, HID), lambda i: (i, 0)),
            pl.BlockSpec((HID, HID), lambda i: (0, 0)),
            pl.BlockSpec((1, HID), lambda i: (0, 0)),
            pl.BlockSpec((HID, HID), lambda i: (0, 0)),
            pl.BlockSpec((HID, HID // 2), lambda i: (0, 0)),
            pl.BlockSpec((1, HID // 2), lambda i: (0, 0)),
            pl.BlockSpec((HID // 2, OUT_DIM), lambda i: (0, 0)),
            pl.BlockSpec((1, OUT_DIM), lambda i: (0, 0)),
        ],
        out_specs=pl.BlockSpec((_BR, OUT_DIM), lambda i: (i, 0)),
        out_shape=jax.ShapeDtypeStruct((NPAD, OUT_DIM), jnp.float32),
    )(agg, cnt, h, wl, bl.reshape(1, -1), wr,
      rw1, rb1.reshape(1, -1), rw2, rb2.reshape(1, -1))


@functools.cache
def _make_sc_kernel(with_count):
    mesh = plsc.VectorSubcoreMesh(core_axis_name="c", subcore_axis_name="s",
                                  num_cores=NC, num_subcores=NS)
    out_type = [jax.ShapeDtypeStruct((NPAD, HID), jnp.float32)]
    if with_count:
        out_type.append(jax.ShapeDtypeStruct((NPAD, CW), jnp.float32))

    scratch = [
        pltpu.VMEM((BB,), jnp.int32),
        pltpu.VMEM((BB,), jnp.int32),
        pltpu.VMEM((BB, CW), jnp.float32),
        pltpu.VMEM((128, CW), jnp.float32),
        pltpu.VMEM_SHARED((NPAD, CW), jnp.float32),
        pltpu.SemaphoreType.DMA,
        pltpu.SemaphoreType.DMA,
    ]

    def body(h8, idx8, dstr, zeros_h, ones_h, *rest):
        if with_count:
            agg_out, cnt_out = rest[0], rest[1]
            idx_v, dst_v, rows_v, ones_v, slab, gsem, ssem = rest[2:]
        else:
            agg_out = rest[0]
            cnt_out = None
            idx_v, dst_v, rows_v, ones_v, slab, gsem, ssem = rest[1:]

        cid = lax.axis_index("c")
        sid = lax.axis_index("s")
        stripe0 = sid * STRIPE

        if with_count:
            pltpu.sync_copy(ones_h, ones_v)

        def zero_slab():
            pltpu.sync_copy(zeros_h, slab.at[pl.ds(stripe0, STRIPE), :])

        def gather_pass(ch):
            zero_slab()
            plsc.subcore_barrier()

            def batch(b, carry):
                e0 = sid * EPT + b * BB
                pltpu.sync_copy(dstr.at[pl.ds(e0, BB)], dst_v)
                pltpu.sync_copy(idx8.at[ch, pl.ds(e0, BB)], idx_v)
                pltpu.async_copy(h8.at[idx_v], rows_v, gsem).wait()
                pltpu.async_copy(rows_v, slab.at[dst_v], ssem,
                                 add=True).wait()
                return carry

            lax.fori_loop(0, NBATCH, batch, 0)
            plsc.subcore_barrier()
            pltpu.sync_copy(
                slab.at[pl.ds(stripe0, STRIPE), :],
                agg_out.at[pl.ds(stripe0, STRIPE), pl.ds(ch * CW, CW)])
            plsc.subcore_barrier()

        for j in range(NCH // NC):
            gather_pass(cid * (NCH // NC) + j)

        if with_count:
            @pl.when(cid == 0)
            def _count_pass():
                zero_slab()
                plsc.subcore_barrier()

                def batch(b, carry):
                    e0 = sid * EPT + b * BB
                    pltpu.sync_copy(dstr.at[pl.ds(e0, BB)], dst_v)
                    for j in range(SUB):
                        pltpu.sync_copy(
                            ones_v,
                            slab.at[dst_v.at[pl.ds(j * 128, 128)]],
                            add=True)
                    return carry

                lax.fori_loop(0, NBATCH, batch, 0)
                plsc.subcore_barrier()
                pltpu.sync_copy(slab.at[pl.ds(stripe0, STRIPE), :],
                                cnt_out.at[pl.ds(stripe0, STRIPE), :])

    return pl.kernel(
        body,
        out_type=tuple(out_type) if with_count else out_type[0],
        mesh=mesh,
        scratch_types=scratch,
        compiler_params=pltpu.CompilerParams(use_tc_tiling_on_sc=False),
    )


def kernel(x, edge_index, enc_w1, enc_b1, enc_w2, enc_b2,
           s1_wl, s1_bl, s1_wr, s2_wl, s2_bl, s2_wr,
           ro_w1, ro_b1, ro_w2, ro_b2):
    xp = jnp.pad(x, ((0, NPAD - N), (0, 0)))

    src = edge_index[0]
    dst = edge_index[1]
    src_p = jnp.concatenate(
        [src, jnp.zeros((EPAD - E,), dtype=jnp.int32)])
    dst_p = jnp.concatenate(
        [dst, jnp.full((EPAD - E,), N, dtype=jnp.int32)])
    idx8 = (src_p[None, :] * NCH
            + jnp.arange(NCH, dtype=jnp.int32)[:, None])
    dstr = dst_p
    zeros_h = jnp.zeros((STRIPE, CW), dtype=jnp.float32)
    ones_h = jnp.ones((128, CW), dtype=jnp.float32)

    h = _encoder(xp, enc_w1, enc_b1, enc_w2, enc_b2)

    agg1, cnt8 = _make_sc_kernel(True)(h.reshape(NPAD * NCH, CW), idx8, dstr,
                                       zeros_h, ones_h)
    cnt = cnt8[:, :1]

    h1 = _combine(agg1, cnt, h, s1_wl, s1_bl, s1_wr)

    agg2 = _make_sc_kernel(False)(h1.reshape(NPAD * NCH, CW), idx8, dstr,
                                  zeros_h, ones_h)

    out = _combine_readout(agg2, cnt, h1, s2_wl, s2_bl, s2_wr,
                           ro_w1, ro_b1, ro_w2, ro_b2)
    return out[:N]


# speedup vs baseline: 3.3417x; 1.3024x over previous
import functools

import jax
import jax.numpy as jnp
from jax import lax
from jax.experimental import pallas as pl
from jax.experimental.pallas import tpu as pltpu
from jax.experimental.pallas import tpu_sc as plsc

N = 50000
NPAD = 50176
IN_DIM = 24
HID = 256
OUT_DIM = 12
E = 800000
EPAD = 819200
NC = 2
NS = 16
CW = 32
NCH = HID // CW
EPT = EPAD // NS
BB = 256
KK = 20
NSUPER = EPT // (BB * KK)
STRIPE = NPAD // NS


def _dot(a, b):
    return lax.dot_general(a, b, (((1,), (0,)), ((), ())),
                           preferred_element_type=jnp.float32)


_BR = 1024
_GRID = NPAD // _BR


def _enc_body(x_ref, w1_ref, b1_ref, w2_ref, b2_ref, o_ref):
    h = jnp.maximum(_dot(x_ref[...], w1_ref[...]) + b1_ref[...], 0.0)
    o_ref[...] = jnp.maximum(_dot(h, w2_ref[...]) + b2_ref[...], 0.0)


def _encoder(xp, w1, b1, w2, b2):
    return pl.pallas_call(
        _enc_body,
        grid=(_GRID,),
        in_specs=[
            pl.BlockSpec((_BR, IN_DIM), lambda i: (i, 0)),
            pl.BlockSpec((IN_DIM, HID // 2), lambda i: (0, 0)),
            pl.BlockSpec((1, HID // 2), lambda i: (0, 0)),
            pl.BlockSpec((HID // 2, HID), lambda i: (0, 0)),
            pl.BlockSpec((1, HID), lambda i: (0, 0)),
        ],
        out_specs=pl.BlockSpec((_BR, HID), lambda i: (i, 0)),
        out_shape=jax.ShapeDtypeStruct((NPAD, HID), jnp.float32),
    )(xp, w1, b1.reshape(1, -1), w2, b2.reshape(1, -1))


def _combine_body(agg_ref, cnt_ref, h_ref, wl_ref, bl_ref, wr_ref, o_ref):
    inv = 1.0 / jnp.maximum(cnt_ref[...], 1.0)
    mean = agg_ref[...] * inv
    o_ref[...] = jnp.maximum(
        _dot(mean, wl_ref[...]) + bl_ref[...] + _dot(h_ref[...], wr_ref[...]),
        0.0)


def _combine(agg, cnt, h, wl, bl, wr):
    return pl.pallas_call(
        _combine_body,
        grid=(_GRID,),
        in_specs=[
            pl.BlockSpec((_BR, HID), lambda i: (i, 0)),
            pl.BlockSpec((_BR, 1), lambda i: (i, 0)),
            pl.BlockSpec((_BR, HID), lambda i: (i, 0)),
            pl.BlockSpec((HID, HID), lambda i: (0, 0)),
            pl.BlockSpec((1, HID), lambda i: (0, 0)),
            pl.BlockSpec((HID, HID), lambda i: (0, 0)),
        ],
        out_specs=pl.BlockSpec((_BR, HID), lambda i: (i, 0)),
        out_shape=jax.ShapeDtypeStruct((NPAD, HID), jnp.float32),
    )(agg, cnt, h, wl, bl.reshape(1, -1), wr)


def _combine_ro_body(agg_ref, cnt_ref, h_ref, wl_ref, bl_ref, wr_ref,
                     rw1_ref, rb1_ref, rw2_ref, rb2_ref, o_ref):
    inv = 1.0 / jnp.maximum(cnt_ref[...], 1.0)
    mean = agg_ref[...] * inv
    t = jnp.maximum(
        _dot(mean, wl_ref[...]) + bl_ref[...] + _dot(h_ref[...], wr_ref[...]),
        0.0)
    t = jnp.maximum(_dot(t, rw1_ref[...]) + rb1_ref[...], 0.0)
    o_ref[...] = _dot(t, rw2_ref[...]) + rb2_ref[...]


def _combine_readout(agg, cnt, h, wl, bl, wr, rw1, rb1, rw2, rb2):
    return pl.pallas_call(
        _combine_ro_body,
        grid=(_GRID,),
        in_specs=[
            pl.BlockSpec((_BR, HID), lambda i: (i, 0)),
            pl.BlockSpec((_BR, 1), lambda i: (i, 0)),
            pl.BlockSpec((_BR, HID), lambda i: (i, 0)),
            pl.BlockSpec((HID, HID), lambda i: (0, 0)),
            pl.BlockSpec((1, HID), lambda i: (0, 0)),
            pl.BlockSpec((HID, HID), lambda i: (0, 0)),
            pl.BlockSpec((HID, HID // 2), lambda i: (0, 0)),
            pl.BlockSpec((1, HID // 2), lambda i: (0, 0)),
            pl.BlockSpec((HID // 2, OUT_DIM), lambda i: (0, 0)),
            pl.BlockSpec((1, OUT_DIM), lambda i: (0, 0)),
        ],
        out_specs=pl.BlockSpec((_BR, OUT_DIM), lambda i: (i, 0)),
        out_shape=jax.ShapeDtypeStruct((NPAD, OUT_DIM), jnp.float32),
    )(agg, cnt, h, wl, bl.reshape(1, -1), wr,
      rw1, rb1.reshape(1, -1), rw2, rb2.reshape(1, -1))


@functools.cache
def _make_sc_kernel(with_count):
    mesh = plsc.VectorSubcoreMesh(core_axis_name="c", subcore_axis_name="s",
                                  num_cores=NC, num_subcores=NS)
    out_type = [jax.ShapeDtypeStruct((NPAD, HID), jnp.float32)]
    if with_count:
        out_type.append(jax.ShapeDtypeStruct((NPAD, CW), jnp.float32))

    scratch = [
        pltpu.VMEM((KK, BB), jnp.int32),
        pltpu.VMEM((KK, BB), jnp.int32),
        pltpu.VMEM((BB, CW), jnp.float32),
        pltpu.VMEM((BB, CW), jnp.float32),
        pltpu.VMEM_SHARED((NPAD, CW), jnp.float32),
        pltpu.SemaphoreType.DMA,
        pltpu.SemaphoreType.DMA,
        pltpu.SemaphoreType.DMA,
        pltpu.SemaphoreType.DMA,
    ]

    def body(h8, idx8, dstr, zeros_h, ones_h, *rest):
        if with_count:
            agg_out, cnt_out = rest[0], rest[1]
            rest = rest[2:]
        else:
            agg_out = rest[0]
            cnt_out = None
            rest = rest[1:]
        idx_all, dst_all, rows0, rows1, slab, g0, g1, s0, s1 = rest
        rows = (rows0, rows1)
        gsem = (g0, g1)
        ssem = (s0, s1)

        cid = lax.axis_index("c")
        sid = lax.axis_index("s")
        stripe0 = sid * STRIPE

        def zero_slab():
            pltpu.sync_copy(zeros_h, slab.at[pl.ds(stripe0, STRIPE), :])

        def gather_pass(ch):
            zero_slab()
            plsc.subcore_barrier()

            def superb(sb, carry):
                row0 = sid * (EPT // BB) + sb * KK
                pltpu.sync_copy(dstr.at[pl.ds(row0, KK), :], dst_all)
                pltpu.sync_copy(idx8.at[ch, pl.ds(row0, KK), :], idx_all)
                gd = [None, None]
                sd = [None, None]
                for k in range(KK):
                    buf = k % 2
                    if sd[buf] is not None:
                        sd[buf].wait()
                    gd[buf] = pltpu.async_copy(
                        h8.at[idx_all.at[k]], rows[buf], gsem[buf])
                    pbuf = 1 - buf
                    if gd[pbuf] is not None:
                        gd[pbuf].wait()
                        sd[pbuf] = pltpu.async_copy(
                            rows[pbuf], slab.at[dst_all.at[k - 1]],
                            ssem[pbuf], add=True)
                lbuf = (KK - 1) % 2
                gd[lbuf].wait()
                sd[lbuf] = pltpu.async_copy(
                    rows[lbuf], slab.at[dst_all.at[KK - 1]],
                    ssem[lbuf], add=True)
                sd[0].wait()
                sd[1].wait()
                return carry

            lax.fori_loop(0, NSUPER, superb, 0)
            plsc.subcore_barrier()
            pltpu.sync_copy(
                slab.at[pl.ds(stripe0, STRIPE), :],
                agg_out.at[pl.ds(stripe0, STRIPE), pl.ds(ch * CW, CW)])
            plsc.subcore_barrier()

        for j in range(NCH // NC):
            gather_pass(cid * (NCH // NC) + j)

        if with_count:
            @pl.when(cid == 0)
            def _count_pass():
                zero_slab()
                pltpu.sync_copy(ones_h, rows0)
                plsc.subcore_barrier()

                def superb(sb, carry):
                    row0 = sid * (EPT // BB) + sb * KK
                    pltpu.sync_copy(dstr.at[pl.ds(row0, KK), :], dst_all)
                    sds = []
                    for k in range(KK):
                        sds.append(pltpu.async_copy(
                            rows0, slab.at[dst_all.at[k]], ssem[0],
                            add=True))
                    for sd in sds:
                        sd.wait()
                    return carry

                lax.fori_loop(0, NSUPER, superb, 0)
                plsc.subcore_barrier()
                pltpu.sync_copy(slab.at[pl.ds(stripe0, STRIPE), :],
                                cnt_out.at[pl.ds(stripe0, STRIPE), :])

    return pl.kernel(
        body,
        out_type=tuple(out_type) if with_count else out_type[0],
        mesh=mesh,
        scratch_types=scratch,
        compiler_params=pltpu.CompilerParams(use_tc_tiling_on_sc=False),
    )


def kernel(x, edge_index, enc_w1, enc_b1, enc_w2, enc_b2,
           s1_wl, s1_bl, s1_wr, s2_wl, s2_bl, s2_wr,
           ro_w1, ro_b1, ro_w2, ro_b2):
    xp = jnp.pad(x, ((0, NPAD - N), (0, 0)))

    src = edge_index[0]
    dst = edge_index[1]
    src_p = jnp.concatenate(
        [src, jnp.zeros((EPAD - E,), dtype=jnp.int32)])
    dst_p = jnp.concatenate(
        [dst, jnp.full((EPAD - E,), N, dtype=jnp.int32)])
    idx8 = (src_p[None, :] * NCH
            + jnp.arange(NCH, dtype=jnp.int32)[:, None]).reshape(
                NCH, EPAD // BB, BB)
    dstr = dst_p.reshape(EPAD // BB, BB)
    zeros_h = jnp.zeros((STRIPE, CW), dtype=jnp.float32)
    ones_h = jnp.ones((BB, CW), dtype=jnp.float32)

    h = _encoder(xp, enc_w1, enc_b1, enc_w2, enc_b2)

    agg1, cnt8 = _make_sc_kernel(True)(h.reshape(NPAD * NCH, CW), idx8, dstr,
                                       zeros_h, ones_h)
    cnt = cnt8[:, :1]

    h1 = _combine(agg1, cnt, h, s1_wl, s1_bl, s1_wr)

    agg2 = _make_sc_kernel(False)(h1.reshape(NPAD * NCH, CW), idx8, dstr,
                                  zeros_h, ones_h)

    out = _combine_readout(agg2, cnt, h1, s2_wl, s2_bl, s2_wr,
                           ro_w1, ro_b1, ro_w2, ro_b2)
    return out[:N]
